# baseline (speedup 1.0000x reference)
import functools

import jax
import jax.numpy as jnp
from jax import lax
from jax.experimental import pallas as pl
from jax.experimental.pallas import tpu as pltpu
from jax.experimental.pallas import tpu_sc as plsc

NC = 2
NS = 16
L = 16
NW = NC * NS
CH = 2
NBUF = 4
TL = 128
TR = 8


def _embed(B, D, labels2, means, stds, noise4):
    BPW = B // NW
    NCHUNK = BPW // CH
    NTC = D // TL

    mesh = plsc.VectorSubcoreMesh(core_axis_name="c", subcore_axis_name="s")

    @functools.partial(
        pl.kernel,
        mesh=mesh,
        out_type=jax.ShapeDtypeStruct((B // TR, NTC, TR, TL), jnp.float32),
        scratch_types=(
            [pltpu.VMEM((NCHUNK, CH), jnp.int32),
             pltpu.VMEM((NBUF, CH, D), jnp.float32),
             pltpu.VMEM((NBUF, CH, D), jnp.float32),
             pltpu.VMEM((NBUF, NTC, CH, TL), jnp.float32)]
            + [pltpu.SemaphoreType.DMA] * (4 * NBUF)
        ),
    )
    def k(labels_hbm, means_hbm, stds_hbm, noise_hbm, out_hbm,
          idx_v, mbuf, sbuf, nbuf, *sems):
        sem_m = sems[0:NBUF]
        sem_s = sems[NBUF:2 * NBUF]
        sem_n = sems[2 * NBUF:3 * NBUF]
        sem_o = sems[3 * NBUF:4 * NBUF]

        wid = lax.axis_index("s") * NC + lax.axis_index("c")
        base = wid * BPW

        pltpu.sync_copy(labels_hbm.at[pl.ds(wid * NCHUNK, NCHUNK)], idx_v)

        def noise_src(c):
            row = base + c * CH
            return noise_hbm.at[row // TR, :, pl.ds(row % TR, CH), :]

        def out_dst(c):
            row = base + c * CH
            return out_hbm.at[row // TR, :, pl.ds(row % TR, CH), :]

        def start_inputs(c, b):
            pltpu.async_copy(means_hbm.at[idx_v.at[c]], mbuf.at[b], sem_m[b])
            pltpu.async_copy(stds_hbm.at[idx_v.at[c]], sbuf.at[b], sem_s[b])
            pltpu.async_copy(noise_src(c), nbuf.at[b], sem_n[b])

        def slot(c, b, first, prefetch):
            pltpu.make_async_copy(
                means_hbm.at[idx_v.at[c]], mbuf.at[b], sem_m[b]).wait()
            pltpu.make_async_copy(
                stds_hbm.at[idx_v.at[c]], sbuf.at[b], sem_s[b]).wait()
            pltpu.make_async_copy(noise_src(c), nbuf.at[b], sem_n[b]).wait()
            for r in range(CH):
                @plsc.parallel_loop(0, NTC, unroll=2)
                def _(tc, r=r, b=b):
                    for kk in range(TL // L):
                        dl = pl.ds(kk * L, L)
                        sl = pl.ds(tc * TL + kk * L, L)
                        nbuf[b, tc, r, dl] = (
                            mbuf[b, r, sl]
                            + sbuf[b, r, sl] * nbuf[b, tc, r, dl])
            pltpu.async_copy(nbuf.at[b], out_dst(c), sem_o[b])
            if prefetch:
                cn = c + (NBUF - 1)
                p = (b + NBUF - 1) % NBUF
                if not first:
                    pltpu.make_async_copy(
                        nbuf.at[p], out_dst(c - 1), sem_o[p]).wait()
                start_inputs(cn, p)

        for b in range(NBUF - 1):
            start_inputs(b, b)

        slot(0, 0, first=True, prefetch=True)

        @pl.loop(0, (NCHUNK - NBUF) // NBUF)
        def _(g):
            c0 = 1 + g * NBUF
            for i in range(NBUF):
                slot(c0 + i, (1 + i) % NBUF, first=False, prefetch=True)

        for i in range(NBUF - 1):
            c = NCHUNK - (NBUF - 1) + i
            slot(c, c % NBUF, first=False, prefetch=False)

        for i in range(NBUF):
            c = NCHUNK - NBUF + i
            b = c % NBUF
            pltpu.make_async_copy(nbuf.at[b], out_dst(c), sem_o[b]).wait()

    return k(labels2, means, stds, noise4)


def kernel(labels, class_means, class_stds, noise):
    num_classes = class_means.shape[0]
    B = labels.shape[0]
    D = class_means.size // num_classes
    noise4 = (noise.reshape(B, D)
              .reshape(B // TR, TR, D // TL, TL)
              .transpose(0, 2, 1, 3))
    out4 = _embed(
        B, D,
        labels.astype(jnp.int32).reshape(B // CH, CH),
        class_means.reshape(num_classes, D),
        class_stds.reshape(num_classes, D),
        noise4,
    )
    out = out4.transpose(0, 2, 1, 3).reshape(B, D)
    return out.reshape(noise.shape)


# speedup vs baseline: 2.0695x; 2.0695x over previous
import functools

import jax
import jax.numpy as jnp
from jax import lax
from jax.experimental import pallas as pl
from jax.experimental.pallas import tpu as pltpu
from jax.experimental.pallas import tpu_sc as plsc

NC = 2
NS = 16
L = 16
NW = NC * NS
CH = 2
NBUF = 4


def _embed(B, D, labels2, means, stds, noise):
    BPW = B // NW
    NCHUNK = BPW // CH

    mesh = plsc.VectorSubcoreMesh(core_axis_name="c", subcore_axis_name="s")

    @functools.partial(
        pl.kernel,
        mesh=mesh,
        out_type=jax.ShapeDtypeStruct((B, D), jnp.float32),
        scratch_types=(
            [pltpu.VMEM((NCHUNK, CH), jnp.int32),
             pltpu.VMEM((NBUF, CH, D), jnp.float32),
             pltpu.VMEM((NBUF, CH, D), jnp.float32),
             pltpu.VMEM((NBUF, CH, D), jnp.float32)]
            + [pltpu.SemaphoreType.DMA] * (4 * NBUF)
        ),
    )
    def k(labels_hbm, means_hbm, stds_hbm, noise_hbm, out_hbm,
          idx_v, mbuf, sbuf, nbuf, *sems):
        sem_m = sems[0:NBUF]
        sem_s = sems[NBUF:2 * NBUF]
        sem_n = sems[2 * NBUF:3 * NBUF]
        sem_o = sems[3 * NBUF:4 * NBUF]

        wid = lax.axis_index("s") * NC + lax.axis_index("c")
        base = wid * BPW

        pltpu.sync_copy(labels_hbm.at[pl.ds(wid * NCHUNK, NCHUNK)], idx_v)

        def start_inputs(c, b):
            pltpu.async_copy(means_hbm.at[idx_v.at[c]], mbuf.at[b], sem_m[b])
            pltpu.async_copy(stds_hbm.at[idx_v.at[c]], sbuf.at[b], sem_s[b])
            pltpu.async_copy(noise_hbm.at[pl.ds(base + c * CH, CH)],
                             nbuf.at[b], sem_n[b])

        def slot(c, b, first, prefetch):
            pltpu.make_async_copy(
                means_hbm.at[idx_v.at[c]], mbuf.at[b], sem_m[b]).wait()
            pltpu.make_async_copy(
                stds_hbm.at[idx_v.at[c]], sbuf.at[b], sem_s[b]).wait()
            pltpu.make_async_copy(
                noise_hbm.at[pl.ds(base + c * CH, CH)], nbuf.at[b],
                sem_n[b]).wait()
            for r in range(CH):
                @plsc.parallel_loop(0, D, step=L, unroll=8)
                def _(j, r=r, b=b):
                    sl = pl.ds(j, L)
                    mbuf[b, r, sl] = (mbuf[b, r, sl]
                                      + sbuf[b, r, sl] * nbuf[b, r, sl])
            pltpu.async_copy(
                mbuf.at[b], out_hbm.at[pl.ds(base + c * CH, CH)], sem_o[b])
            if prefetch:
                cn = c + (NBUF - 1)
                p = (b + NBUF - 1) % NBUF
                if not first:
                    pltpu.make_async_copy(
                        mbuf.at[p],
                        out_hbm.at[pl.ds(base + (c - 1) * CH, CH)],
                        sem_o[p]).wait()
                start_inputs(cn, p)

        for b in range(NBUF - 1):
            start_inputs(b, b)

        slot(0, 0, first=True, prefetch=True)

        @pl.loop(0, (NCHUNK - NBUF) // NBUF)
        def _(g):
            c0 = 1 + g * NBUF
            for i in range(NBUF):
                slot(c0 + i, (1 + i) % NBUF, first=False, prefetch=True)

        for i in range(NBUF - 1):
            c = NCHUNK - (NBUF - 1) + i
            slot(c, c % NBUF, first=False, prefetch=False)

        for i in range(NBUF):
            c = NCHUNK - NBUF + i
            b = c % NBUF
            pltpu.make_async_copy(
                mbuf.at[b], out_hbm.at[pl.ds(base + c * CH, CH)],
                sem_o[b]).wait()

    return k(labels2, means, stds, noise)


def kernel(labels, class_means, class_stds, noise):
    num_classes = class_means.shape[0]
    B = labels.shape[0]
    D = class_means.size // num_classes
    means2 = class_means.reshape(num_classes, D)
    stds2 = class_stds.reshape(num_classes, D)
    noise2 = noise.reshape(B, D)
    labels2 = labels.astype(jnp.int32).reshape(B // CH, CH)
    NSPLIT = 4
    Bs = B // NSPLIT
    outs = []
    for i in range(NSPLIT):
        outs.append(_embed(
            Bs, D,
            labels2[i * (Bs // CH):(i + 1) * (Bs // CH)],
            means2, stds2,
            noise2[i * Bs:(i + 1) * Bs],
        ))
    return jnp.concatenate(outs, axis=0).reshape(noise.shape)


# speedup vs baseline: 2.7436x; 1.3257x over previous
import functools

import jax
import jax.numpy as jnp
from jax import lax
from jax.experimental import pallas as pl
from jax.experimental.pallas import tpu as pltpu
from jax.experimental.pallas import tpu_sc as plsc

NC = 2
NS = 16
L = 16
NW = NC * NS
CH = 2
NBUF = 4


def _embed(B, D, labels2, means, stds, noise):
    BPW = B // NW
    NCHUNK = BPW // CH

    mesh = plsc.VectorSubcoreMesh(core_axis_name="c", subcore_axis_name="s")

    @functools.partial(
        pl.kernel,
        mesh=mesh,
        out_type=jax.ShapeDtypeStruct((B, D), jnp.float32),
        scratch_types=(
            [pltpu.VMEM((NCHUNK, CH), jnp.int32),
             pltpu.VMEM((NBUF, CH, D), jnp.float32),
             pltpu.VMEM((NBUF, CH, D), jnp.float32),
             pltpu.VMEM((NBUF, CH, D), jnp.float32)]
            + [pltpu.SemaphoreType.DMA] * (4 * NBUF)
        ),
    )
    def k(labels_hbm, means_hbm, stds_hbm, noise_hbm, out_hbm,
          idx_v, mbuf, sbuf, nbuf, *sems):
        sem_m = sems[0:NBUF]
        sem_s = sems[NBUF:2 * NBUF]
        sem_n = sems[2 * NBUF:3 * NBUF]
        sem_o = sems[3 * NBUF:4 * NBUF]

        wid = lax.axis_index("s") * NC + lax.axis_index("c")
        base = wid * BPW

        pltpu.sync_copy(labels_hbm.at[pl.ds(wid * NCHUNK, NCHUNK)], idx_v)

        def start_inputs(c, b):
            pltpu.async_copy(means_hbm.at[idx_v.at[c]], mbuf.at[b], sem_m[b])
            pltpu.async_copy(stds_hbm.at[idx_v.at[c]], sbuf.at[b], sem_s[b])
            pltpu.async_copy(noise_hbm.at[pl.ds(base + c * CH, CH)],
                             nbuf.at[b], sem_n[b])

        def slot(c, b, first, prefetch):
            pltpu.make_async_copy(
                means_hbm.at[idx_v.at[c]], mbuf.at[b], sem_m[b]).wait()
            pltpu.make_async_copy(
                stds_hbm.at[idx_v.at[c]], sbuf.at[b], sem_s[b]).wait()
            pltpu.make_async_copy(
                noise_hbm.at[pl.ds(base + c * CH, CH)], nbuf.at[b],
                sem_n[b]).wait()
            for r in range(CH):
                @plsc.parallel_loop(0, D, step=L, unroll=8)
                def _(j, r=r, b=b):
                    sl = pl.ds(j, L)
                    mbuf[b, r, sl] = (mbuf[b, r, sl]
                                      + sbuf[b, r, sl] * nbuf[b, r, sl])
            pltpu.async_copy(
                mbuf.at[b], out_hbm.at[pl.ds(base + c * CH, CH)], sem_o[b])
            if prefetch:
                cn = c + (NBUF - 1)
                p = (b + NBUF - 1) % NBUF
                if not first:
                    pltpu.make_async_copy(
                        mbuf.at[p],
                        out_hbm.at[pl.ds(base + (c - 1) * CH, CH)],
                        sem_o[p]).wait()
                start_inputs(cn, p)

        for b in range(NBUF - 1):
            start_inputs(b, b)

        slot(0, 0, first=True, prefetch=True)

        @pl.loop(0, (NCHUNK - NBUF) // NBUF)
        def _(g):
            c0 = 1 + g * NBUF
            for i in range(NBUF):
                slot(c0 + i, (1 + i) % NBUF, first=False, prefetch=True)

        for i in range(NBUF - 1):
            c = NCHUNK - (NBUF - 1) + i
            slot(c, c % NBUF, first=False, prefetch=False)

        for i in range(NBUF):
            c = NCHUNK - NBUF + i
            b = c % NBUF
            pltpu.make_async_copy(
                mbuf.at[b], out_hbm.at[pl.ds(base + c * CH, CH)],
                sem_o[b]).wait()

    return k(labels2, means, stds, noise)


def kernel(labels, class_means, class_stds, noise):
    num_classes = class_means.shape[0]
    B = labels.shape[0]
    D = class_means.size // num_classes
    out = _embed(
        B, D,
        labels.astype(jnp.int32).reshape(B // CH, CH),
        class_means.reshape(num_classes, D),
        class_stds.reshape(num_classes, D),
        noise.reshape(B, D),
    )
    return out.reshape(noise.shape)
